# baseline (speedup 1.0000x reference)
import functools

import jax
import jax.numpy as jnp
from jax import lax
from jax.experimental import pallas as pl
from jax.experimental.pallas import tpu as pltpu
from jax.experimental.pallas import tpu_sc as plsc

DIM = 64
CUTOFF = 5.0
N_CENTERS = 50
L = 16
NC = 2
NS = 16
NW = NC * NS

_INTERPRET = False


def _softplus(x, beta=0.5, threshold=14.0):
    return jnp.where(beta * x > threshold, x,
                     (1.0 / beta) * jnp.log1p(jnp.exp(jnp.minimum(beta * x, threshold))))


TROWS = 168
TSCALE = 32.0


def _tc_table(edge_mask, w1, b1, w2, b2):
    gap = CUTOFF / (N_CENTERS - 1)

    def body(em_ref, w1_ref, b1_ref, w2_ref, b2_ref, o_ref):
        ri = lax.broadcasted_iota(jnp.int32, (TROWS, 1), 0)
        d2 = ri.astype(jnp.float32) * (1.0 / TSCALE)
        centers = lax.broadcasted_iota(
            jnp.int32, (1, N_CENTERS), 1).astype(jnp.float32) * gap
        rbf = jnp.exp((-1.0 / gap) * (d2 - centers) ** 2)
        rbf = jnp.where(ri >= 161, em_ref[...][None, :], rbf)
        hh = _softplus(jnp.dot(rbf, w1_ref[...], preferred_element_type=jnp.float32)
                       + b1_ref[...][None, :])
        o_ref[...] = (jnp.dot(hh, w2_ref[...], preferred_element_type=jnp.float32)
                      + b2_ref[...][None, :])

    return pl.pallas_call(
        body,
        out_shape=jax.ShapeDtypeStruct((TROWS, DIM), jnp.float32),
        interpret=_INTERPRET,
    )(edge_mask, w1, b1, w2, b2)


def _tc_matmul(x, w, tile=1024):
    n = x.shape[0]

    def body(x_ref, w_ref, o_ref):
        o_ref[...] = jnp.dot(x_ref[...], w_ref[...], preferred_element_type=jnp.float32)

    return pl.pallas_call(
        body,
        grid=(n // tile,),
        in_specs=[pl.BlockSpec((tile, DIM), lambda i: (i, 0)),
                  pl.BlockSpec((DIM, DIM), lambda i: (0, 0))],
        out_specs=pl.BlockSpec((tile, DIM), lambda i: (i, 0)),
        out_shape=jax.ShapeDtypeStruct((n, DIM), jnp.float32),
        interpret=_INTERPRET,
    )(x, w)


def _tc_update(node, agg, w2, b2, w3, b3, tile=1024):
    n = node.shape[0]

    def body(n_ref, a_ref, w2_ref, b2_ref, w3_ref, b3_ref, o_ref):
        cf = _softplus(jnp.dot(a_ref[...], w2_ref[...], preferred_element_type=jnp.float32)
                       + b2_ref[...][None, :])
        o_ref[...] = n_ref[...] + (
            jnp.dot(cf, w3_ref[...], preferred_element_type=jnp.float32)
            + b3_ref[...][None, :])

    return pl.pallas_call(
        body,
        grid=(n // tile,),
        in_specs=[pl.BlockSpec((tile, DIM), lambda i: (i, 0)),
                  pl.BlockSpec((tile, DIM), lambda i: (i, 0)),
                  pl.BlockSpec((DIM, DIM), lambda i: (0, 0)),
                  pl.BlockSpec((DIM,), lambda i: (0,)),
                  pl.BlockSpec((DIM, DIM), lambda i: (0, 0)),
                  pl.BlockSpec((DIM,), lambda i: (0,))],
        out_specs=pl.BlockSpec((tile, DIM), lambda i: (i, 0)),
        out_shape=jax.ShapeDtypeStruct((n, DIM), jnp.float32),
        interpret=_INTERPRET,
    )(node, agg, w2, b2, w3, b3)


def _tc_heads(nsel, ssel, tsel, wn1, bn1, wn2p, bn2p, we1s, we1t, be1, we2p, be2p,
              tile=512):
    n = nsel.shape[0]

    def body(ns_ref, ss_ref, ts_ref, wn1_ref, bn1_ref, wn2_ref, bn2_ref,
             we1s_ref, we1t_ref, be1_ref, we2_ref, be2_ref, nt_ref, et_ref):
        f32 = jnp.float32
        t1 = jnp.dot(ns_ref[...], wn1_ref[...], preferred_element_type=f32) + bn1_ref[...][None, :]
        nt_ref[...] = jnp.dot(t1, wn2_ref[...], preferred_element_type=f32) + bn2_ref[...][None, :]
        e1 = (jnp.dot(ss_ref[...], we1s_ref[...], preferred_element_type=f32)
              + jnp.dot(ts_ref[...], we1t_ref[...], preferred_element_type=f32)
              + be1_ref[...][None, :])
        et_ref[...] = jnp.dot(e1, we2_ref[...], preferred_element_type=f32) + be2_ref[...][None, :]

    full = lambda *shape: pl.BlockSpec(shape, lambda i: tuple(0 for _ in shape))
    return pl.pallas_call(
        body,
        grid=(n // tile,),
        in_specs=[pl.BlockSpec((tile, DIM), lambda i: (i, 0)),
                  pl.BlockSpec((tile, DIM), lambda i: (i, 0)),
                  pl.BlockSpec((tile, DIM), lambda i: (i, 0)),
                  full(DIM, 32), full(32,), full(32, 8), full(8,),
                  full(DIM, DIM), full(DIM, DIM), full(DIM,), full(DIM, 8), full(8,)],
        out_specs=[pl.BlockSpec((tile, 8), lambda i: (i, 0)),
                   pl.BlockSpec((tile, 8), lambda i: (i, 0))],
        out_shape=[jax.ShapeDtypeStruct((n, 8), jnp.float32),
                   jax.ShapeDtypeStruct((n, 8), jnp.float32)],
        interpret=_INTERPRET,
    )(nsel, ssel, tsel, wn1, bn1, wn2p, bn2p, we1s, we1t, be1, we2p, be2p)


def _sc_prologue(node_type_p, embedding, dist_p, dst_p, sel_p):
    npad = node_type_p.shape[0]
    epad = dist_p.shape[0]
    selpad = sel_p.shape[0]
    half = epad // NC
    rows_t = npad // NW
    n_nch = rows_t // 112
    d_t = half // NS
    sel_t = selpad // NW
    n_sch = sel_t // 128

    mesh = plsc.VectorSubcoreMesh(core_axis_name="c", subcore_axis_name="s",
                                  num_cores=NC, num_subcores=NS)

    npad_n = npad
    e_t = epad // NS

    @functools.partial(
        pl.kernel,
        out_type=[jax.ShapeDtypeStruct((npad, DIM), jnp.float32),
                  jax.ShapeDtypeStruct((epad,), jnp.float32),
                  jax.ShapeDtypeStruct((NW, 16), jnp.int32)],
        mesh=mesh,
        compiler_params=pltpu.CompilerParams(use_tc_tiling_on_sc=False, needs_layout_passes=False),
        scratch_types=[
            pltpu.VMEM((112,), jnp.int32),
            pltpu.VMEM((112, DIM), jnp.float32),
            pltpu.VMEM((128,), jnp.int32),
            pltpu.VMEM((128,), jnp.int32),
            pltpu.VMEM((128,), jnp.float32),
            pltpu.VMEM((128,), jnp.int32),
            pltpu.VMEM((16,), jnp.int32),
            pltpu.VMEM_SHARED((half + 8,), jnp.float32),
            pltpu.SemaphoreType.DMA,
        ],
        interpret=_INTERPRET,
    )
    def k(nt_hbm, emb_hbm, dist_hbm, dstp_hbm, sel_hbm, node_hbm, dm_hbm, cnt_hbm,
          idx_v, rows_v, sel_v, lsel_v, neg_v, dchunk_v, row_v, stage, sem):
        c = lax.axis_index("c")
        s = lax.axis_index("s")
        wid = s * NC + c
        cbase = c * half
        nbase = c * (npad_n // NC)
        nhalf_n = npad_n // NC

        def cnt_body(j, acc):
            pltpu.sync_copy(dstp_hbm.at[pl.ds(s * e_t + j * 128, 128)], dchunk_v)
            for g in range(8):
                dv = dchunk_v[pl.ds(g * 16, 16)]
                inr = (dv >= nbase) & (dv < nbase + nhalf_n)
                acc = acc + jnp.where(inr, 1, 0).astype(jnp.int32)
            return acc

        acc16 = lax.fori_loop(0, e_t // 128, cnt_body,
                              jnp.zeros((16,), jnp.int32))
        cnt = jnp.sum(acc16)
        lane = lax.broadcasted_iota(jnp.int32, (16,), 0)
        row_v[...] = jnp.where(lane == 0, cnt, 0)
        pltpu.sync_copy(row_v, cnt_hbm.at[c * NS + s])

        pltpu.sync_copy(dist_hbm.at[pl.ds(cbase + s * d_t, d_t)],
                        stage.at[pl.ds(s * d_t, d_t)])
        plsc.subcore_barrier()

        for g in range(8):
            neg_v[pl.ds(g * 16, 16)] = jnp.full((16,), -1.0, jnp.float32)

        def sel_body(j, _):
            pltpu.sync_copy(sel_hbm.at[pl.ds(wid * sel_t + j * 128, 128)], sel_v)
            for g in range(8):
                sv = sel_v[pl.ds(g * 16, 16)]
                inr = (sv >= cbase) & (sv < cbase + half)
                lsel_v[pl.ds(g * 16, 16)] = jnp.where(
                    inr, sv - cbase, jnp.full((16,), half, jnp.int32))
            pltpu.sync_copy(neg_v, stage.at[lsel_v])
            return 0

        lax.fori_loop(0, n_sch, sel_body, 0)
        plsc.subcore_barrier()

        pltpu.sync_copy(stage.at[pl.ds(s * d_t, d_t)],
                        dm_hbm.at[pl.ds(cbase + s * d_t, d_t)])

        def emb_body(j, _):
            base = wid * rows_t + j * 112
            pltpu.sync_copy(nt_hbm.at[pl.ds(base, 112)], idx_v)
            pltpu.async_copy(emb_hbm.at[idx_v], rows_v, sem).wait()
            pltpu.sync_copy(rows_v, node_hbm.at[pl.ds(base, 112)])
            return 0

        lax.fori_loop(0, n_nch, emb_body, 0)

    return k(node_type_p, embedding, dist_p, dst_p, sel_p)


def _run_offsets(cnt_hbm, cnts_v, c, s):
    pltpu.sync_copy(cnt_hbm, cnts_v)
    lane = lax.broadcasted_iota(jnp.int32, (16,), 0)
    zeros = jnp.zeros((16,), jnp.int32)
    cnt0 = plsc.load_gather(cnts_v, [lane, zeros])
    cnt1 = plsc.load_gather(cnts_v, [lane + 16, zeros])
    len0 = jnp.maximum(((cnt0 + 127) >> 7) << 7, 128)
    len1 = jnp.maximum(((cnt1 + 127) >> 7) << 7, 128)
    cum0 = plsc.cumsum(len0)
    cum1 = plsc.cumsum(len1)
    total0 = jnp.sum(jnp.where(lane == 15, cum0, 0))
    ofs_v = jnp.where(c == 0, cum0 - len0, cum1 - len1 + total0)
    len_v = jnp.where(c == 0, len0, len1)
    ofs = jnp.sum(jnp.where(lane == s, ofs_v, 0))
    ln = jnp.sum(jnp.where(lane == s, len_v, 0))
    return ofs, ln


def _sc_partition(dm, src_p, dst_p, counts, npad):
    epad = dm.shape[0]
    epad2 = epad + NW * 128
    e_t = epad // NS
    nhalf = npad // NC

    mesh = plsc.VectorSubcoreMesh(core_axis_name="c", subcore_axis_name="s",
                                  num_cores=NC, num_subcores=NS)

    @functools.partial(
        pl.kernel,
        out_type=[jax.ShapeDtypeStruct((epad2,), jnp.int32),
                  jax.ShapeDtypeStruct((epad2,), jnp.int32),
                  jax.ShapeDtypeStruct((epad2,), jnp.int32),
                  jax.ShapeDtypeStruct((epad2,), jnp.float32)],
        mesh=mesh,
        compiler_params=pltpu.CompilerParams(use_tc_tiling_on_sc=False, needs_layout_passes=False),
        scratch_types=[
            pltpu.VMEM((128,), jnp.int32),
            pltpu.VMEM((128,), jnp.int32),
            pltpu.VMEM((128,), jnp.float32),
            pltpu.VMEM((160,), jnp.int32),
            pltpu.VMEM((160,), jnp.int32),
            pltpu.VMEM((160,), jnp.int32),
            pltpu.VMEM((160,), jnp.float32),
            pltpu.VMEM((NW, 16), jnp.int32),
        ],
        interpret=_INTERPRET,
    )
    def k(dm_hbm, src_hbm, dst_hbm, cnt_hbm, psrc_hbm, pldst_hbm, pidx_hbm, pfrac_hbm,
          sbuf, dbuf, tbuf, psrc, pldst, pidx, pfrac, cnts_v):
        c = lax.axis_index("c")
        s = lax.axis_index("s")
        cbase = c * nhalf
        ofs, _ = _run_offsets(cnt_hbm, cnts_v, c, s)

        def chunk_body(j, carry):
            pos, cursor = carry
            base = s * e_t + j * 128
            pltpu.sync_copy(src_hbm.at[pl.ds(base, 128)], sbuf)
            pltpu.sync_copy(dst_hbm.at[pl.ds(base, 128)], dbuf)
            pltpu.sync_copy(dm_hbm.at[pl.ds(base, 128)], tbuf)
            for g in range(8):
                s16 = sbuf[pl.ds(g * 16, 16)]
                d16 = dbuf[pl.ds(g * 16, 16)]
                t16 = tbuf[pl.ds(g * 16, 16)]
                m = (d16 >= cbase) & (d16 < cbase + nhalf)
                mi = jnp.where(m, 1, 0).astype(jnp.int32)
                cum = plsc.cumsum(mi)
                idx = pos + cum - 1
                tt = t16 * TSCALE
                it = tt.astype(jnp.int32)
                i16 = jnp.where(t16 < 0.0, 161, it)
                f16 = jnp.where(t16 < 0.0, 0.0, tt - it.astype(jnp.float32))
                plsc.store_scatter(psrc, [idx], s16, mask=m)
                plsc.store_scatter(pldst, [idx], d16 - cbase, mask=m)
                plsc.store_scatter(pidx, [idx], i16, mask=m)
                plsc.store_scatter(pfrac, [idx], f16, mask=m)
                pos = pos + jnp.sum(mi)
                do_flush = pos >= 128

                def _fl():
                    cur = pl.multiple_of(cursor, 128)
                    pltpu.sync_copy(psrc.at[pl.ds(0, 128)],
                                    psrc_hbm.at[pl.ds(cur, 128)])
                    pltpu.sync_copy(pldst.at[pl.ds(0, 128)],
                                    pldst_hbm.at[pl.ds(cur, 128)])
                    pltpu.sync_copy(pidx.at[pl.ds(0, 128)],
                                    pidx_hbm.at[pl.ds(cur, 128)])
                    pltpu.sync_copy(pfrac.at[pl.ds(0, 128)],
                                    pfrac_hbm.at[pl.ds(cur, 128)])
                    psrc[pl.ds(0, 16)] = psrc[pl.ds(128, 16)]
                    pldst[pl.ds(0, 16)] = pldst[pl.ds(128, 16)]
                    pidx[pl.ds(0, 16)] = pidx[pl.ds(128, 16)]
                    pfrac[pl.ds(0, 16)] = pfrac[pl.ds(128, 16)]

                pl.when(do_flush)(_fl)
                pos = jnp.where(do_flush, pos - 128, pos)
                cursor = jnp.where(do_flush, cursor + 128, cursor)
            return (pos, cursor)

        pos, cursor = lax.fori_loop(0, e_t // 128, chunk_body,
                                    (jnp.int32(0), ofs))

        cond_emit = (pos > 0) | (cursor == ofs)
        fsrc = jnp.zeros((16,), jnp.int32)
        fldst = jnp.full((16,), nhalf, jnp.int32)
        fidx = jnp.zeros((16,), jnp.int32)
        ffrac = jnp.zeros((16,), jnp.float32)
        lane = lax.broadcasted_iota(jnp.int32, (16,), 0)
        for kk in range(8):
            idxf = pos + kk * 16 + lane
            mf = (idxf < 128) & cond_emit
            plsc.store_scatter(psrc, [idxf], fsrc, mask=mf)
            plsc.store_scatter(pldst, [idxf], fldst, mask=mf)
            plsc.store_scatter(pidx, [idxf], fidx, mask=mf)
            plsc.store_scatter(pfrac, [idxf], ffrac, mask=mf)

        def _ff():
            cur = pl.multiple_of(cursor, 128)
            pltpu.sync_copy(psrc.at[pl.ds(0, 128)], psrc_hbm.at[pl.ds(cur, 128)])
            pltpu.sync_copy(pldst.at[pl.ds(0, 128)], pldst_hbm.at[pl.ds(cur, 128)])
            pltpu.sync_copy(pidx.at[pl.ds(0, 128)], pidx_hbm.at[pl.ds(cur, 128)])
            pltpu.sync_copy(pfrac.at[pl.ds(0, 128)], pfrac_hbm.at[pl.ds(cur, 128)])

        pl.when(cond_emit)(_ff)

    return k(dm, src_p, dst_p, counts)


def _sc_msg(part_src, part_ldst, part_idx, part_frac, counts, new_node, table):
    npad = new_node.shape[0]
    nhalf = npad // NC
    ch = 128
    w_rows = nhalf // NS

    mesh = plsc.VectorSubcoreMesh(core_axis_name="c", subcore_axis_name="s",
                                  num_cores=NC, num_subcores=NS)

    @functools.partial(
        pl.kernel,
        out_type=jax.ShapeDtypeStruct((npad, DIM), jnp.float32),
        mesh=mesh,
        compiler_params=pltpu.CompilerParams(use_tc_tiling_on_sc=False,
                                             needs_layout_passes=False),
        scratch_types=(
            [pltpu.VMEM((ch,), jnp.int32)] * 2
            + [pltpu.VMEM((ch,), jnp.int32)] * 2
            + [pltpu.VMEM((ch,), jnp.int32)] * 2
            + [pltpu.VMEM((ch,), jnp.float32)] * 2
            + [pltpu.VMEM((ch, DIM), jnp.float32)] * 2
            + [pltpu.VMEM((TROWS, DIM), jnp.float32)]
            + [pltpu.VMEM((NW, 16), jnp.int32)]
            + [pltpu.SemaphoreType.DMA] * 6
            + [pltpu.VMEM_SHARED((nhalf + 8, DIM), jnp.float32)]
        ),
        interpret=_INTERPRET,
    )
    def k(src_hbm, ldst_hbm, idx_hbm, frac_hbm, cnt_hbm, nn_hbm, tab_hbm, agg_hbm,
          src0, src1, ld0, ld1, ix0, ix1, fr0, fr1, nn0, nn1, tab_v, cnts_v,
          ls0, ls1, gs0, gs1, ss0, ss1, acc):
        src_v = [src0, src1]
        ldst_v = [ld0, ld1]
        idx_v = [ix0, ix1]
        frac_v = [fr0, fr1]
        nn_v = [nn0, nn1]
        lsem = [ls0, ls1]
        gsem = [gs0, gs1]
        ssem = [ss0, ss1]
        c = lax.axis_index("c")
        s = lax.axis_index("s")
        cbase = c * nhalf
        ofs, ln = _run_offsets(cnt_hbm, cnts_v, c, s)
        ofs = pl.multiple_of(ofs, 128)
        nch = ln >> 7

        pltpu.sync_copy(tab_hbm, tab_v)

        def zfill(r, _):
            for q in range(DIM // 16):
                nn0[r, pl.ds(q * 16, 16)] = jnp.zeros((16,), jnp.float32)
            return 0

        lax.fori_loop(0, ch, zfill, 0)

        def zero_body(j, _):
            pltpu.sync_copy(nn0, acc.at[pl.ds(s * w_rows + j * ch, ch)])
            return 0

        lax.fori_loop(0, w_rows // ch, zero_body, 0)
        rem = w_rows % ch
        if rem:
            pltpu.sync_copy(nn0.at[pl.ds(0, rem)],
                            acc.at[pl.ds(s * w_rows + (w_rows // ch) * ch, rem)])
        pl.when(s == 0)(lambda: pltpu.sync_copy(
            nn0.at[pl.ds(0, 8)], acc.at[pl.ds(nhalf, 8)]))
        plsc.subcore_barrier()

        def ebase(j):
            return ofs + j * ch

        def issue_loads(j, b):
            pltpu.async_copy(src_hbm.at[pl.ds(ebase(j), ch)], src_v[b], lsem[b])
            pltpu.async_copy(ldst_hbm.at[pl.ds(ebase(j), ch)], ldst_v[b], lsem[b])
            pltpu.async_copy(idx_hbm.at[pl.ds(ebase(j), ch)], idx_v[b], lsem[b])
            pltpu.async_copy(frac_hbm.at[pl.ds(ebase(j), ch)], frac_v[b], lsem[b])

        def wait_loads(j, b):
            pltpu.make_async_copy(src_hbm.at[pl.ds(ebase(j), ch)], src_v[b], lsem[b]).wait()
            pltpu.make_async_copy(ldst_hbm.at[pl.ds(ebase(j), ch)], ldst_v[b], lsem[b]).wait()
            pltpu.make_async_copy(idx_hbm.at[pl.ds(ebase(j), ch)], idx_v[b], lsem[b]).wait()
            pltpu.make_async_copy(frac_hbm.at[pl.ds(ebase(j), ch)], frac_v[b], lsem[b]).wait()

        def issue_gather(b):
            pltpu.async_copy(nn_hbm.at[src_v[b]], nn_v[b], gsem[b])

        def wait_gather(b):
            pltpu.make_async_copy(nn_hbm.at[src_v[b]], nn_v[b], gsem[b]).wait()

        def issue_scatter(b):
            pltpu.async_copy(nn_v[b], acc.at[ldst_v[b]], ssem[b], add=True)

        def wait_scatter(b):
            pltpu.make_async_copy(nn_v[b], acc.at[ldst_v[b]], ssem[b]).wait()

        issue_loads(0, 0)
        wait_loads(0, 0)
        issue_gather(0)

        lane = lax.broadcasted_iota(jnp.int32, (16,), 0)

        def edge_body(jj, _):
            for b in range(2):
                j = jj * 2 + b
                ob = 1 - b

                def step():
                    pl.when(j + 1 < nch)(lambda: issue_loads(j + 1, ob))
                    wait_gather(b)

                    def grp_body(g, _):
                        e16 = g * 16 + lane
                        iv = idx_v[b][pl.ds(g * 16, 16)]
                        ivp = iv + 1
                        fv = frac_v[b][pl.ds(g * 16, 16)]
                        for col in range(DIM):
                            cc = jnp.full((16,), col, jnp.int32)
                            tlo = plsc.load_gather(tab_v, [iv, cc])
                            thi = plsc.load_gather(tab_v, [ivp, cc])
                            nn16 = plsc.load_gather(nn_v[b], [e16, cc])
                            h16 = tlo + fv * (thi - tlo)
                            plsc.store_scatter(nn_v[b], [e16, cc], nn16 * h16)
                        return 0

                    lax.fori_loop(0, ch // 16, grp_body, 0)
                    issue_scatter(b)

                    def finish_next():
                        wait_loads(j + 1, ob)
                        pl.when(j >= 1)(lambda: wait_scatter(ob))
                        issue_gather(ob)

                    pl.when(j + 1 < nch)(finish_next)

                if b == 0:
                    step()
                else:
                    pl.when(j < nch)(step)
            return 0

        lax.fori_loop(0, (nch + 1) >> 1, edge_body, 0)
        wait_scatter(0)
        pl.when(nch > 1)(lambda: wait_scatter(1))
        plsc.subcore_barrier()

        pltpu.sync_copy(acc.at[pl.ds(s * w_rows, w_rows)],
                        agg_hbm.at[pl.ds(cbase + s * w_rows, w_rows)])

    return k(part_src, part_ldst, part_idx, part_frac, counts, new_node, table)


def _sc_gather3(feature, ni_p, si_p, ti_p):
    nsel = ni_p.shape[0]
    per_w = nsel // NW
    n_ch = per_w // 128

    mesh = plsc.VectorSubcoreMesh(core_axis_name="c", subcore_axis_name="s",
                                  num_cores=NC, num_subcores=NS)

    @functools.partial(
        pl.kernel,
        out_type=[jax.ShapeDtypeStruct((nsel, DIM), jnp.float32)] * 3,
        mesh=mesh,
        compiler_params=pltpu.CompilerParams(use_tc_tiling_on_sc=False, needs_layout_passes=False),
        scratch_types=[
            pltpu.VMEM((128,), jnp.int32),
            pltpu.VMEM((128, DIM), jnp.float32),
            pltpu.SemaphoreType.DMA,
        ],
        interpret=_INTERPRET,
    )
    def k(f_hbm, ni_hbm, si_hbm, ti_hbm, no_hbm, so_hbm, to_hbm, idx_v, rows_v, sem):
        c = lax.axis_index("c")
        s = lax.axis_index("s")
        wid = s * NC + c

        def gather_one(idx_hbm, out_hbm):
            def body(j, _):
                base = wid * per_w + j * 128
                pltpu.sync_copy(idx_hbm.at[pl.ds(base, 128)], idx_v)
                pltpu.async_copy(f_hbm.at[idx_v], rows_v, sem).wait()
                pltpu.sync_copy(rows_v, out_hbm.at[pl.ds(base, 128)])
                return 0
            lax.fori_loop(0, n_ch, body, 0)

        gather_one(ni_hbm, no_hbm)
        gather_one(si_hbm, so_hbm)
        gather_one(ti_hbm, to_hbm)

    return k(feature, ni_p, si_p, ti_p)


def kernel(node_type, edge_index, distance, node_index, source_index, target_index,
           select_edge_index, embedding, edge_mask, conv_params,
           W_nt1, b_nt1, W_nt2, b_nt2, W_et1, b_et1, W_et2, b_et2):
    n = node_type.shape[0]
    e = distance.shape[0]
    nsel = node_index.shape[0]
    esel = select_edge_index.shape[0]

    def rup(x, m):
        return ((x + m - 1) // m) * m

    npad = rup(n, NW * 112)
    epad = rup(e, NS * 256)
    selpad = rup(esel, NW * 128)
    nselpad = rup(nsel, NW * 128)

    nt_p = jnp.pad(node_type, (0, npad - n))
    src_p = jnp.pad(edge_index[0], (0, epad - e))
    dst_p = jnp.pad(edge_index[1], (0, epad - e), constant_values=1 << 30)
    dist_p = jnp.pad(distance, (0, epad - e))
    sel_p = jnp.pad(select_edge_index, (0, selpad - esel), constant_values=epad)
    ni_p = jnp.pad(node_index, (0, nselpad - nsel))
    si_p = jnp.pad(source_index, (0, nselpad - nsel))
    ti_p = jnp.pad(target_index, (0, nselpad - nsel))

    node, dist_m, counts = _sc_prologue(nt_p, embedding, dist_p, dst_p, sel_p)
    part_src, part_ldst, part_idx, part_frac = _sc_partition(
        dist_m, src_p, dst_p, counts, npad)

    for p in conv_params:
        nn = _tc_matmul(node, p["W_nl1"])
        tab = _tc_table(edge_mask, p["W_cf1"], p["b_cf1"], p["W_cf2"], p["b_cf2"])
        agg = _sc_msg(part_src, part_ldst, part_idx, part_frac, counts, nn, tab)
        node = _tc_update(node, agg, p["W_nl2"], p["b_nl2"], p["W_nl3"], p["b_nl3"])

    nrows, srows, trows = _sc_gather3(node, ni_p, si_p, ti_p)

    wn2p = jnp.pad(W_nt2, ((0, 0), (0, 8 - W_nt2.shape[1])))
    bn2p = jnp.pad(b_nt2, (0, 8 - b_nt2.shape[0]))
    we2p = jnp.pad(W_et2, ((0, 0), (0, 8 - W_et2.shape[1])))
    be2p = jnp.pad(b_et2, (0, 8 - b_et2.shape[0]))
    nt8, et8 = _tc_heads(nrows, srows, trows, W_nt1, b_nt1, wn2p, bn2p,
                         W_et1[:DIM], W_et1[DIM:], b_et1, we2p, be2p)
    return nt8[:nsel, :3], et8[:nsel, :5]


# speedup vs baseline: 3.2154x; 3.2154x over previous
import functools

import jax
import jax.numpy as jnp
from jax import lax
from jax.experimental import pallas as pl
from jax.experimental.pallas import tpu as pltpu
from jax.experimental.pallas import tpu_sc as plsc

DIM = 64
CUTOFF = 5.0
N_CENTERS = 50
L = 16
NC = 2
NS = 16
NW = NC * NS

_INTERPRET = False


def _softplus(x, beta=0.5, threshold=14.0):
    return jnp.where(beta * x > threshold, x,
                     (1.0 / beta) * jnp.log1p(jnp.exp(jnp.minimum(beta * x, threshold))))


def _tc_h(dist_m, edge_mask, w1, b1, w2, b2, tile=1024):
    epad = dist_m.shape[0]
    gap = CUTOFF / (N_CENTERS - 1)

    def body(d_ref, em_ref, w1_ref, b1_ref, w2_ref, b2_ref, o_ref):
        d2 = d_ref[...][:, None]
        centers = lax.broadcasted_iota(
            jnp.int32, (1, N_CENTERS), 1).astype(jnp.float32) * gap
        rbf = jnp.exp((-1.0 / gap) * (d2 - centers) ** 2)
        rbf = jnp.where(d2 < 0.0, em_ref[...][None, :], rbf)
        hh = _softplus(jnp.dot(rbf, w1_ref[...], preferred_element_type=jnp.float32)
                       + b1_ref[...][None, :])
        o_ref[...] = (jnp.dot(hh, w2_ref[...], preferred_element_type=jnp.float32)
                      + b2_ref[...][None, :])

    return pl.pallas_call(
        body,
        grid=(epad // tile,),
        in_specs=[
            pl.BlockSpec((tile,), lambda i: (i,)),
            pl.BlockSpec((N_CENTERS,), lambda i: (0,)),
            pl.BlockSpec((N_CENTERS, DIM), lambda i: (0, 0)),
            pl.BlockSpec((DIM,), lambda i: (0,)),
            pl.BlockSpec((DIM, DIM), lambda i: (0, 0)),
            pl.BlockSpec((DIM,), lambda i: (0,)),
        ],
        out_specs=pl.BlockSpec((tile, DIM), lambda i: (i, 0)),
        out_shape=jax.ShapeDtypeStruct((epad, DIM), jnp.float32),
        interpret=_INTERPRET,
    )(dist_m, edge_mask, w1, b1, w2, b2)


def _tc_matmul(x, w, tile=1024):
    n = x.shape[0]

    def body(x_ref, w_ref, o_ref):
        o_ref[...] = jnp.dot(x_ref[...], w_ref[...], preferred_element_type=jnp.float32)

    return pl.pallas_call(
        body,
        grid=(n // tile,),
        in_specs=[pl.BlockSpec((tile, DIM), lambda i: (i, 0)),
                  pl.BlockSpec((DIM, DIM), lambda i: (0, 0))],
        out_specs=pl.BlockSpec((tile, DIM), lambda i: (i, 0)),
        out_shape=jax.ShapeDtypeStruct((n, DIM), jnp.float32),
        interpret=_INTERPRET,
    )(x, w)


def _tc_update(node, agg, w2, b2, w3, b3, tile=1024):
    n = node.shape[0]

    def body(n_ref, a_ref, w2_ref, b2_ref, w3_ref, b3_ref, o_ref):
        cf = _softplus(jnp.dot(a_ref[...], w2_ref[...], preferred_element_type=jnp.float32)
                       + b2_ref[...][None, :])
        o_ref[...] = n_ref[...] + (
            jnp.dot(cf, w3_ref[...], preferred_element_type=jnp.float32)
            + b3_ref[...][None, :])

    return pl.pallas_call(
        body,
        grid=(n // tile,),
        in_specs=[pl.BlockSpec((tile, DIM), lambda i: (i, 0)),
                  pl.BlockSpec((tile, DIM), lambda i: (i, 0)),
                  pl.BlockSpec((DIM, DIM), lambda i: (0, 0)),
                  pl.BlockSpec((DIM,), lambda i: (0,)),
                  pl.BlockSpec((DIM, DIM), lambda i: (0, 0)),
                  pl.BlockSpec((DIM,), lambda i: (0,))],
        out_specs=pl.BlockSpec((tile, DIM), lambda i: (i, 0)),
        out_shape=jax.ShapeDtypeStruct((n, DIM), jnp.float32),
        interpret=_INTERPRET,
    )(node, agg, w2, b2, w3, b3)


def _tc_heads(nsel, ssel, tsel, wn1, bn1, wn2p, bn2p, we1s, we1t, be1, we2p, be2p,
              tile=512):
    n = nsel.shape[0]

    def body(ns_ref, ss_ref, ts_ref, wn1_ref, bn1_ref, wn2_ref, bn2_ref,
             we1s_ref, we1t_ref, be1_ref, we2_ref, be2_ref, nt_ref, et_ref):
        f32 = jnp.float32
        t1 = jnp.dot(ns_ref[...], wn1_ref[...], preferred_element_type=f32) + bn1_ref[...][None, :]
        nt_ref[...] = jnp.dot(t1, wn2_ref[...], preferred_element_type=f32) + bn2_ref[...][None, :]
        e1 = (jnp.dot(ss_ref[...], we1s_ref[...], preferred_element_type=f32)
              + jnp.dot(ts_ref[...], we1t_ref[...], preferred_element_type=f32)
              + be1_ref[...][None, :])
        et_ref[...] = jnp.dot(e1, we2_ref[...], preferred_element_type=f32) + be2_ref[...][None, :]

    full = lambda *shape: pl.BlockSpec(shape, lambda i: tuple(0 for _ in shape))
    return pl.pallas_call(
        body,
        grid=(n // tile,),
        in_specs=[pl.BlockSpec((tile, DIM), lambda i: (i, 0)),
                  pl.BlockSpec((tile, DIM), lambda i: (i, 0)),
                  pl.BlockSpec((tile, DIM), lambda i: (i, 0)),
                  full(DIM, 32), full(32,), full(32, 8), full(8,),
                  full(DIM, DIM), full(DIM, DIM), full(DIM,), full(DIM, 8), full(8,)],
        out_specs=[pl.BlockSpec((tile, 8), lambda i: (i, 0)),
                   pl.BlockSpec((tile, 8), lambda i: (i, 0))],
        out_shape=[jax.ShapeDtypeStruct((n, 8), jnp.float32),
                   jax.ShapeDtypeStruct((n, 8), jnp.float32)],
        interpret=_INTERPRET,
    )(nsel, ssel, tsel, wn1, bn1, wn2p, bn2p, we1s, we1t, be1, we2p, be2p)


def _sc_prologue(node_type_p, embedding, dist_p, sel_p):
    npad = node_type_p.shape[0]
    epad = dist_p.shape[0]
    selpad = sel_p.shape[0]
    half = epad // NC
    rows_t = npad // NW
    n_nch = rows_t // 112
    d_t = half // NS
    sel_t = selpad // NW
    n_sch = sel_t // 128

    mesh = plsc.VectorSubcoreMesh(core_axis_name="c", subcore_axis_name="s",
                                  num_cores=NC, num_subcores=NS)

    @functools.partial(
        pl.kernel,
        out_type=[jax.ShapeDtypeStruct((npad, DIM), jnp.float32),
                  jax.ShapeDtypeStruct((epad,), jnp.float32)],
        mesh=mesh,
        compiler_params=pltpu.CompilerParams(use_tc_tiling_on_sc=False,
                                             needs_layout_passes=False),
        scratch_types=[
            pltpu.VMEM((112,), jnp.int32),
            pltpu.VMEM((112, DIM), jnp.float32),
            pltpu.VMEM((128,), jnp.int32),
            pltpu.VMEM((128,), jnp.int32),
            pltpu.VMEM((128,), jnp.float32),
            pltpu.VMEM_SHARED((half + 8,), jnp.float32),
            pltpu.SemaphoreType.DMA,
        ],
        interpret=_INTERPRET,
    )
    def k(nt_hbm, emb_hbm, dist_hbm, sel_hbm, node_hbm, dm_hbm,
          idx_v, rows_v, sel_v, lsel_v, neg_v, stage, sem):
        c = lax.axis_index("c")
        s = lax.axis_index("s")
        wid = s * NC + c
        cbase = c * half

        pltpu.sync_copy(dist_hbm.at[pl.ds(cbase + s * d_t, d_t)],
                        stage.at[pl.ds(s * d_t, d_t)])
        plsc.subcore_barrier()

        for g in range(8):
            neg_v[pl.ds(g * 16, 16)] = jnp.full((16,), -1.0, jnp.float32)

        def sel_body(j, _):
            pltpu.sync_copy(sel_hbm.at[pl.ds(wid * sel_t + j * 128, 128)], sel_v)
            for g in range(8):
                sv = sel_v[pl.ds(g * 16, 16)]
                inr = (sv >= cbase) & (sv < cbase + half)
                lsel_v[pl.ds(g * 16, 16)] = jnp.where(
                    inr, sv - cbase, jnp.full((16,), half, jnp.int32))
            pltpu.sync_copy(neg_v, stage.at[lsel_v])
            return 0

        lax.fori_loop(0, n_sch, sel_body, 0)
        plsc.subcore_barrier()

        pltpu.sync_copy(stage.at[pl.ds(s * d_t, d_t)],
                        dm_hbm.at[pl.ds(cbase + s * d_t, d_t)])

        def emb_body(j, _):
            base = wid * rows_t + j * 112
            pltpu.sync_copy(nt_hbm.at[pl.ds(base, 112)], idx_v)
            pltpu.async_copy(emb_hbm.at[idx_v], rows_v, sem).wait()
            pltpu.sync_copy(rows_v, node_hbm.at[pl.ds(base, 112)])
            return 0

        lax.fori_loop(0, n_nch, emb_body, 0)

    return k(node_type_p, embedding, dist_p, sel_p)


def _sc_msg(h, src_p, dst_p, new_node):
    epad = h.shape[0]
    npad = new_node.shape[0]
    nhalf = npad // NC
    e_t = epad // NS
    ch = 128
    n_ech = e_t // ch
    w_rows = nhalf // NS

    mesh = plsc.VectorSubcoreMesh(core_axis_name="c", subcore_axis_name="s",
                                  num_cores=NC, num_subcores=NS)

    @functools.partial(
        pl.kernel,
        out_type=jax.ShapeDtypeStruct((npad, DIM), jnp.float32),
        mesh=mesh,
        compiler_params=pltpu.CompilerParams(use_tc_tiling_on_sc=False,
                                             needs_layout_passes=False),
        scratch_types=(
            [pltpu.VMEM((ch,), jnp.int32)] * 2
            + [pltpu.VMEM((ch,), jnp.int32)] * 2
            + [pltpu.VMEM((ch,), jnp.int32)] * 2
            + [pltpu.VMEM((ch, DIM), jnp.float32)] * 2
            + [pltpu.VMEM((ch, DIM), jnp.float32)]
            + [pltpu.SemaphoreType.DMA] * 7
            + [pltpu.VMEM_SHARED((nhalf + 8, DIM), jnp.float32)]
        ),
        interpret=_INTERPRET,
    )
    def k(h_hbm, src_hbm, dst_hbm, nn_hbm, agg_hbm,
          src0, src1, dst0, dst1, ld0, ld1, nn0, nn1, hv,
          ls0, ls1, hs, gs0, gs1, ss0, ss1, acc):
        src_v = [src0, src1]
        dst_v = [dst0, dst1]
        ldst_v = [ld0, ld1]
        nn_v = [nn0, nn1]
        lsem = [ls0, ls1]
        gsem = [gs0, gs1]
        ssem = [ss0, ss1]
        c = lax.axis_index("c")
        s = lax.axis_index("s")
        cbase = c * nhalf

        def zfill(r, _):
            for q in range(DIM // 16):
                hv[r, pl.ds(q * 16, 16)] = jnp.zeros((16,), jnp.float32)
            return 0

        lax.fori_loop(0, ch, zfill, 0)

        def zero_body(j, _):
            pltpu.sync_copy(hv, acc.at[pl.ds(s * w_rows + j * ch, ch)])
            return 0

        lax.fori_loop(0, w_rows // ch, zero_body, 0)
        rem = w_rows % ch
        if rem:
            pltpu.sync_copy(hv.at[pl.ds(0, rem)],
                            acc.at[pl.ds(s * w_rows + (w_rows // ch) * ch, rem)])
        pl.when(s == 0)(lambda: pltpu.sync_copy(
            hv.at[pl.ds(0, 8)], acc.at[pl.ds(nhalf, 8)]))
        plsc.subcore_barrier()

        def ebase(j):
            return s * e_t + j * ch

        def issue_sd(j, b):
            pltpu.async_copy(src_hbm.at[pl.ds(ebase(j), ch)], src_v[b], lsem[b])
            pltpu.async_copy(dst_hbm.at[pl.ds(ebase(j), ch)], dst_v[b], lsem[b])

        def wait_sd(j, b):
            pltpu.make_async_copy(src_hbm.at[pl.ds(ebase(j), ch)], src_v[b], lsem[b]).wait()
            pltpu.make_async_copy(dst_hbm.at[pl.ds(ebase(j), ch)], dst_v[b], lsem[b]).wait()

        def issue_h(j):
            pltpu.async_copy(h_hbm.at[pl.ds(ebase(j), ch)], hv, hs)

        def wait_h(j):
            pltpu.make_async_copy(h_hbm.at[pl.ds(ebase(j), ch)], hv, hs).wait()

        def issue_gather(b):
            pltpu.async_copy(nn_hbm.at[src_v[b]], nn_v[b], gsem[b])

        def wait_gather(b):
            pltpu.make_async_copy(nn_hbm.at[src_v[b]], nn_v[b], gsem[b]).wait()

        def issue_scatter(b):
            pltpu.async_copy(nn_v[b], acc.at[ldst_v[b]], ssem[b], add=True)

        def wait_scatter(b):
            pltpu.make_async_copy(nn_v[b], acc.at[ldst_v[b]], ssem[b]).wait()

        issue_sd(0, 0)
        issue_h(0)
        wait_sd(0, 0)
        issue_gather(0)

        def edge_body(jj, _):
            for b in range(2):
                j = jj * 2 + b
                ob = 1 - b

                pl.when(j + 1 < n_ech)(lambda: issue_sd(j + 1, ob))

                for g in range(ch // 16):
                    dv = dst_v[b][pl.ds(g * 16, 16)]
                    inr = (dv >= cbase) & (dv < cbase + nhalf)
                    ldst_v[b][pl.ds(g * 16, 16)] = jnp.where(
                        inr, dv - cbase, jnp.full((16,), nhalf, jnp.int32))
                wait_h(j)
                wait_gather(b)

                def mul_body(m, _):
                    for ee in range(4):
                        e = m * 4 + ee
                        for q in range(DIM // 16):
                            nn_v[b][e, pl.ds(q * 16, 16)] = (
                                nn_v[b][e, pl.ds(q * 16, 16)]
                                * hv[e, pl.ds(q * 16, 16)])
                    return 0

                lax.fori_loop(0, ch // 4, mul_body, 0)
                issue_scatter(b)

                def finish_next():
                    issue_h(j + 1)
                    wait_sd(j + 1, ob)
                    pl.when(j >= 1)(lambda: wait_scatter(ob))
                    issue_gather(ob)

                pl.when(j + 1 < n_ech)(finish_next)
            return 0

        lax.fori_loop(0, n_ech // 2, edge_body, 0)
        wait_scatter(0)
        wait_scatter(1)
        plsc.subcore_barrier()

        pltpu.sync_copy(acc.at[pl.ds(s * w_rows, w_rows)],
                        agg_hbm.at[pl.ds(cbase + s * w_rows, w_rows)])

    return k(h, src_p, dst_p, new_node)


def _sc_gather3(feature, ni_p, si_p, ti_p):
    nsel = ni_p.shape[0]
    per_w = nsel // NW
    n_ch = per_w // 128

    mesh = plsc.VectorSubcoreMesh(core_axis_name="c", subcore_axis_name="s",
                                  num_cores=NC, num_subcores=NS)

    @functools.partial(
        pl.kernel,
        out_type=[jax.ShapeDtypeStruct((nsel, DIM), jnp.float32)] * 3,
        mesh=mesh,
        compiler_params=pltpu.CompilerParams(use_tc_tiling_on_sc=False, needs_layout_passes=False),
        scratch_types=[
            pltpu.VMEM((128,), jnp.int32),
            pltpu.VMEM((128, DIM), jnp.float32),
            pltpu.SemaphoreType.DMA,
        ],
        interpret=_INTERPRET,
    )
    def k(f_hbm, ni_hbm, si_hbm, ti_hbm, no_hbm, so_hbm, to_hbm, idx_v, rows_v, sem):
        c = lax.axis_index("c")
        s = lax.axis_index("s")
        wid = s * NC + c

        def gather_one(idx_hbm, out_hbm):
            def body(j, _):
                base = wid * per_w + j * 128
                pltpu.sync_copy(idx_hbm.at[pl.ds(base, 128)], idx_v)
                pltpu.async_copy(f_hbm.at[idx_v], rows_v, sem).wait()
                pltpu.sync_copy(rows_v, out_hbm.at[pl.ds(base, 128)])
                return 0
            lax.fori_loop(0, n_ch, body, 0)

        gather_one(ni_hbm, no_hbm)
        gather_one(si_hbm, so_hbm)
        gather_one(ti_hbm, to_hbm)

    return k(feature, ni_p, si_p, ti_p)


def kernel(node_type, edge_index, distance, node_index, source_index, target_index,
           select_edge_index, embedding, edge_mask, conv_params,
           W_nt1, b_nt1, W_nt2, b_nt2, W_et1, b_et1, W_et2, b_et2):
    n = node_type.shape[0]
    e = distance.shape[0]
    nsel = node_index.shape[0]
    esel = select_edge_index.shape[0]

    def rup(x, m):
        return ((x + m - 1) // m) * m

    npad = rup(n, NW * 112)
    epad = rup(e, NS * 256)
    selpad = rup(esel, NW * 128)
    nselpad = rup(nsel, NW * 128)

    nt_p = jnp.pad(node_type, (0, npad - n))
    src_p = jnp.pad(edge_index[0], (0, epad - e))
    dst_p = jnp.pad(edge_index[1], (0, epad - e), constant_values=npad)
    dist_p = jnp.pad(distance, (0, epad - e))
    sel_p = jnp.pad(select_edge_index, (0, selpad - esel), constant_values=epad)
    ni_p = jnp.pad(node_index, (0, nselpad - nsel))
    si_p = jnp.pad(source_index, (0, nselpad - nsel))
    ti_p = jnp.pad(target_index, (0, nselpad - nsel))

    node, dist_m = _sc_prologue(nt_p, embedding, dist_p, sel_p)

    for p in conv_params:
        nn = _tc_matmul(node, p["W_nl1"])
        h = _tc_h(dist_m, edge_mask, p["W_cf1"], p["b_cf1"], p["W_cf2"], p["b_cf2"])
        agg = _sc_msg(h, src_p, dst_p, nn)
        node = _tc_update(node, agg, p["W_nl2"], p["b_nl2"], p["W_nl3"], p["b_nl3"])

    nrows, srows, trows = _sc_gather3(node, ni_p, si_p, ti_p)

    wn2p = jnp.pad(W_nt2, ((0, 0), (0, 8 - W_nt2.shape[1])))
    bn2p = jnp.pad(b_nt2, (0, 8 - b_nt2.shape[0]))
    we2p = jnp.pad(W_et2, ((0, 0), (0, 8 - W_et2.shape[1])))
    be2p = jnp.pad(b_et2, (0, 8 - b_et2.shape[0]))
    nt8, et8 = _tc_heads(nrows, srows, trows, W_nt1, b_nt1, wn2p, bn2p,
                         W_et1[:DIM], W_et1[DIM:], b_et1, we2p, be2p)
    return nt8[:nsel, :3], et8[:nsel, :5]
